# baseline (speedup 1.0000x reference)
import jax
import jax.numpy as jnp
from jax.experimental import pallas as pl

N, S, K = 4096, 200, 100
K_BLK = 5
GRID = K // K_BLK


def _body(in_ref, out_ref):
    ids = in_ref[...]
    k0 = pl.program_id(0) * K_BLK
    for kk in range(K_BLK):
        out_ref[kk] = (ids == (k0 + kk)).astype(jnp.float32)


def kernel(inputs):
    x_t = inputs.T
    out_t = pl.pallas_call(
        _body,
        grid=(GRID,),
        in_specs=[pl.BlockSpec((S, N), lambda i: (0, 0))],
        out_specs=pl.BlockSpec((K_BLK, S, N), lambda i: (i, 0, 0)),
        out_shape=jax.ShapeDtypeStruct((K, S, N), jnp.float32),
    )(x_t)
    return jnp.transpose(out_t, (2, 1, 0))


# speedup vs baseline: 1.0020x; 1.0020x over previous
import jax
import jax.numpy as jnp
from jax import lax
from jax.experimental import pallas as pl
from jax.experimental.pallas import tpu as pltpu

N, S, K = 4096, 200, 100
K_BLK = 4
GRID = K // K_BLK
NBUF = 3


def _body(in_ref, out_hbm, buf, sems):
    i = pl.program_id(0)
    slot = lax.rem(i, NBUF)

    @pl.when(i >= NBUF)
    def _wait_prev():
        pltpu.make_async_copy(
            buf.at[slot],
            out_hbm.at[pl.ds((i - NBUF) * K_BLK, K_BLK)],
            sems.at[slot],
        ).wait()

    ids = in_ref[...]
    k0 = i * K_BLK
    for kk in range(K_BLK):
        oh = (ids == (k0 + kk)).astype(jnp.float32)
        buf[pl.ds(slot, 1), pl.ds(kk, 1)] = oh.reshape(1, 1, S, N)

    pltpu.make_async_copy(
        buf.at[slot],
        out_hbm.at[pl.ds(i * K_BLK, K_BLK)],
        sems.at[slot],
    ).start()

    @pl.when(i == GRID - 1)
    def _drain():
        for j in range(NBUF):
            pltpu.make_async_copy(
                buf.at[j],
                out_hbm.at[pl.ds(0, K_BLK)],
                sems.at[j],
            ).wait()


def kernel(inputs):
    x_t = inputs.T
    out_t = pl.pallas_call(
        _body,
        grid=(GRID,),
        in_specs=[pl.BlockSpec((S, N), lambda i: (0, 0))],
        out_specs=pl.BlockSpec(memory_space=pl.ANY),
        out_shape=jax.ShapeDtypeStruct((K, S, N), jnp.float32),
        scratch_shapes=[
            pltpu.VMEM((NBUF, K_BLK, S, N), jnp.float32),
            pltpu.SemaphoreType.DMA((NBUF,)),
        ],
    )(x_t)
    return jnp.transpose(out_t, (2, 1, 0))
